# baseline (speedup 1.0000x reference)
import functools

import jax
import jax.numpy as jnp
from jax import lax
from jax.experimental import pallas as pl
from jax.experimental.pallas import tpu as pltpu
from jax.experimental.pallas import tpu_sc as plsc


def _seg_body(idx_ref, seg_ref):
    idx = idx_ref[...]
    seg_ref[...] = idx[0] * 2 + idx[1]


def _combine_body(part_ref, adj_ref, outz_ref, out_ref):
    sums = jnp.sum(part_ref[...], axis=0)
    adj = adj_ref[...]
    deg = jnp.maximum(jnp.sum((adj >= 0).astype(jnp.float32), axis=3), 1.0)
    out_ref[...] = outz_ref[...]
    out_ref[:, 0:2, :] = sums[0:4, :].reshape(2, 2, 128) / deg


def _make_sc_call(N, d, out_rows, nw, rows_w, chunk, vrows_w, zrows):
    mesh = plsc.VectorSubcoreMesh(core_axis_name="c", subcore_axis_name="s")
    nc = 2

    @functools.partial(
        pl.kernel,
        mesh=mesh,
        out_type=[
            jax.ShapeDtypeStruct((out_rows, d), jnp.float32),
            jax.ShapeDtypeStruct((nw, 8, d), jnp.float32),
        ],
        scratch_types=[
            pltpu.VMEM((chunk, d), jnp.float32),
            pltpu.VMEM((chunk + 16,), jnp.int32),
            pltpu.VMEM((8, d), jnp.float32),
            pltpu.VMEM((zrows, d), jnp.float32),
        ],
    )
    def sc_call(seg_hbm, feat_hbm, out_hbm, part_hbm, feat_v, seg_v, acc_v, zero_v):
        wid = lax.axis_index("s") * nc + lax.axis_index("c")
        z16 = jnp.zeros((16,), jnp.float32)
        for i in range(8):
            for j in range(d // 16):
                acc_v[i, pl.ds(j * 16, 16)] = z16

        def zrow(i, carry):
            for j in range(d // 16):
                zero_v[i, pl.ds(j * 16, 16)] = z16
            return carry

        lax.fori_loop(0, zrows, zrow, 0)

        zslab = 624

        def zfill(k, carry):
            pltpu.sync_copy(
                zero_v, out_hbm.at[pl.ds(wid * zslab + k * zrows, zrows)]
            )
            return carry

        lax.fori_loop(0, zslab // zrows, zfill, 0)

        @pl.when(wid == 0)
        def _tail():
            pltpu.sync_copy(
                zero_v.at[pl.ds(0, out_rows - nw * zslab)],
                out_hbm.at[pl.ds(nw * zslab, out_rows - nw * zslab)],
            )

        def chunk_body(g, carry):
            base = wid * rows_w + g * chunk
            pltpu.sync_copy(feat_hbm.at[pl.ds(base, chunk)], feat_v)
            pltpu.sync_copy(seg_hbm.at[pl.ds(base, chunk)], seg_v.at[pl.ds(0, chunk)])

            def row_body(r, c2):
                s = seg_v[pl.ds(r, 16)][0]
                for j in range(d // 16):
                    sl = pl.ds(j * 16, 16)
                    acc_v[s, sl] = acc_v[s, sl] + feat_v[r, sl]
                return c2

            lax.fori_loop(0, chunk, row_body, 0)
            return carry

        lax.fori_loop(0, rows_w // chunk, chunk_body, 0)
        pltpu.sync_copy(acc_v, part_hbm.at[wid])

    return sc_call


def kernel(adjacency, flattened_indices_0, flattened_features_0):
    B, V, T, S = adjacency.shape
    N, d = flattened_features_0.shape
    out_rows = B * V
    nw = 32
    rows_w = N // nw
    chunk = 200
    vrows_w = out_rows // nw
    zrows = 208

    idx_3 = flattened_indices_0.T.reshape(3, N // 128, 128)

    seg2d = pl.pallas_call(
        _seg_body,
        grid=(1,),
        in_specs=[pl.BlockSpec((3, N // 128, 128), lambda i: (0, 0, 0))],
        out_specs=pl.BlockSpec((N // 128, 128), lambda i: (0, 0)),
        out_shape=jax.ShapeDtypeStruct((N // 128, 128), jnp.int32),
    )(idx_3)
    seg_flat = seg2d.reshape(N)

    sc_call = _make_sc_call(N, d, out_rows, nw, rows_w, chunk, vrows_w, zrows)
    out_flat, part = sc_call(seg_flat, flattened_features_0)
    out_zeros = out_flat.reshape(B, V, d)

    out = pl.pallas_call(
        _combine_body,
        grid=(1,),
        in_specs=[
            pl.BlockSpec((nw, 8, d), lambda i: (0, 0, 0)),
            pl.BlockSpec((B, 2, T, S), lambda i: (0, 0, 0, 0)),
            pl.BlockSpec((B, 8, d), lambda i: (0, 0, 0)),
        ],
        out_specs=pl.BlockSpec((B, 8, d), lambda i: (0, 0, 0)),
        out_shape=jax.ShapeDtypeStruct((B, V, d), jnp.float32),
        input_output_aliases={2: 0},
    )(part, adjacency, out_zeros)
    return out


# speedup vs baseline: 2.2306x; 2.2306x over previous
import functools

import jax
import jax.numpy as jnp
from jax import lax
from jax.experimental import pallas as pl
from jax.experimental.pallas import tpu as pltpu
from jax.experimental.pallas import tpu_sc as plsc


def _seg_body(idx_ref, seg_ref):
    idx = idx_ref[...]
    seg_ref[...] = idx[0] * 2 + idx[1]


def _combine_body(part_ref, adj_ref, outz_ref, out_ref):
    sums = jnp.sum(part_ref[...], axis=0)
    adj = adj_ref[...]
    deg = jnp.maximum(jnp.sum((adj >= 0).astype(jnp.float32), axis=3), 1.0)
    out_ref[...] = outz_ref[...]
    out_ref[:, 0:2, :] = sums[0:4, :].reshape(2, 2, 128) / deg


def _make_sc_call(N, d, out_rows, nw, rows_w, chunk, vrows_w, zrows):
    mesh = plsc.VectorSubcoreMesh(core_axis_name="c", subcore_axis_name="s")
    nc = 2

    @functools.partial(
        pl.kernel,
        mesh=mesh,
        out_type=[
            jax.ShapeDtypeStruct((out_rows, d), jnp.float32),
            jax.ShapeDtypeStruct((nw, 8, d), jnp.float32),
        ],
        scratch_types=[
            pltpu.VMEM((chunk, d), jnp.float32),
            pltpu.VMEM((chunk + 16,), jnp.int32),
            pltpu.VMEM((8, d), jnp.float32),
            pltpu.VMEM((zrows, d), jnp.float32),
        ],
    )
    def sc_call(seg_hbm, feat_hbm, out_hbm, part_hbm, feat_v, seg_v, acc_v, zero_v):
        wid = lax.axis_index("s") * nc + lax.axis_index("c")
        z16 = jnp.zeros((16,), jnp.float32)
        for i in range(8):
            for j in range(d // 16):
                acc_v[i, pl.ds(j * 16, 16)] = z16

        def zrow(i, carry):
            for j in range(d // 16):
                zero_v[i, pl.ds(j * 16, 16)] = z16
            return carry

        lax.fori_loop(0, zrows, zrow, 0)

        zslab = 624

        def zfill(k, carry):
            pltpu.sync_copy(
                zero_v, out_hbm.at[pl.ds(wid * zslab + k * zrows, zrows)]
            )
            return carry

        lax.fori_loop(0, zslab // zrows, zfill, 0)

        @pl.when(wid == 0)
        def _tail():
            pltpu.sync_copy(
                zero_v.at[pl.ds(0, out_rows - nw * zslab)],
                out_hbm.at[pl.ds(nw * zslab, out_rows - nw * zslab)],
            )

        nj = d // 16
        accs0 = tuple(
            jnp.zeros((16,), jnp.float32) for _ in range(4 * nj)
        )

        def add_rows(accs, r0, nrows):
            for k in range(nrows):
                r = r0 + k
                s = seg_v[pl.ds(r, 16)][0]
                row = tuple(feat_v[r, pl.ds(j * 16, 16)] for j in range(nj))
                accs = list(accs)
                for sg in range(4):
                    m = jnp.where(s == sg, 1.0, 0.0).astype(jnp.float32)
                    for j in range(nj):
                        accs[sg * nj + j] = accs[sg * nj + j] + row[j] * m
                accs = tuple(accs)
            return accs

        def chunk_body(g, accs):
            base = wid * rows_w + g * chunk
            pltpu.sync_copy(feat_hbm.at[pl.ds(base, chunk)], feat_v)
            pltpu.sync_copy(seg_hbm.at[pl.ds(base, chunk)], seg_v.at[pl.ds(0, chunk)])

            def grp_body(t, a):
                return add_rows(a, t * 16, 16)

            accs = lax.fori_loop(0, chunk // 16, grp_body, accs)
            return add_rows(accs, (chunk // 16) * 16, chunk % 16)

        accs = lax.fori_loop(0, rows_w // chunk, chunk_body, accs0)
        for sg in range(4):
            for j in range(nj):
                acc_v[sg, pl.ds(j * 16, 16)] = accs[sg * nj + j]
        pltpu.sync_copy(acc_v, part_hbm.at[wid])

    return sc_call


def kernel(adjacency, flattened_indices_0, flattened_features_0):
    B, V, T, S = adjacency.shape
    N, d = flattened_features_0.shape
    out_rows = B * V
    nw = 32
    rows_w = N // nw
    chunk = 200
    vrows_w = out_rows // nw
    zrows = 208

    idx_3 = flattened_indices_0.T.reshape(3, N // 128, 128)

    seg2d = pl.pallas_call(
        _seg_body,
        grid=(1,),
        in_specs=[pl.BlockSpec((3, N // 128, 128), lambda i: (0, 0, 0))],
        out_specs=pl.BlockSpec((N // 128, 128), lambda i: (0, 0)),
        out_shape=jax.ShapeDtypeStruct((N // 128, 128), jnp.int32),
    )(idx_3)
    seg_flat = seg2d.reshape(N)

    sc_call = _make_sc_call(N, d, out_rows, nw, rows_w, chunk, vrows_w, zrows)
    out_flat, part = sc_call(seg_flat, flattened_features_0)
    out_zeros = out_flat.reshape(B, V, d)

    out = pl.pallas_call(
        _combine_body,
        grid=(1,),
        in_specs=[
            pl.BlockSpec((nw, 8, d), lambda i: (0, 0, 0)),
            pl.BlockSpec((B, 2, T, S), lambda i: (0, 0, 0, 0)),
            pl.BlockSpec((B, 8, d), lambda i: (0, 0, 0)),
        ],
        out_specs=pl.BlockSpec((B, 8, d), lambda i: (0, 0, 0)),
        out_shape=jax.ShapeDtypeStruct((B, V, d), jnp.float32),
        input_output_aliases={2: 0},
    )(part, adjacency, out_zeros)
    return out
